# baseline (speedup 1.0000x reference)
import jax
import jax.numpy as jnp
from jax import lax
from jax.experimental import pallas as pl
from jax.experimental.pallas import tpu as pltpu
from jax.experimental.pallas import tpu_sc as plsc

_BATCH = 16384
_EMB = 64
_NC = 2
_NS = 16
_NW = _NC * _NS
_BPW = _BATCH // _NW


def _gather_body(user_table, item_table, user_ids, item_ids,
                 u_out, i_out, uidx, urows, iidx, irows, usem, isem):
    wid = lax.axis_index("s") * _NC + lax.axis_index("c")
    base = wid * _BPW
    pltpu.sync_copy(user_ids.at[pl.ds(base, _BPW)], uidx)
    pltpu.sync_copy(item_ids.at[pl.ds(base, _BPW)], iidx)
    ucp = pltpu.async_copy(user_table.at[uidx], urows, usem)
    icp = pltpu.async_copy(item_table.at[iidx], irows, isem)
    ucp.wait()
    pltpu.sync_copy(urows, u_out.at[pl.ds(base, _BPW)])
    icp.wait()
    pltpu.sync_copy(irows, i_out.at[pl.ds(base, _BPW)])


_gather_cache = []


def _gather_kernel():
    if not _gather_cache:
        _gather_cache.append(pl.kernel(
            _gather_body,
            out_type=[jax.ShapeDtypeStruct((_BATCH, _EMB), jnp.float32),
                      jax.ShapeDtypeStruct((_BATCH, _EMB), jnp.float32)],
            mesh=plsc.VectorSubcoreMesh(core_axis_name="c",
                                        subcore_axis_name="s",
                                        num_cores=_NC, num_subcores=_NS),
            scratch_types=[
                pltpu.VMEM((_BPW,), jnp.int32),
                pltpu.VMEM((_BPW, _EMB), jnp.float32),
                pltpu.VMEM((_BPW,), jnp.int32),
                pltpu.VMEM((_BPW, _EMB), jnp.float32),
                pltpu.SemaphoreType.DMA,
                pltpu.SemaphoreType.DMA,
            ],
            compiler_params=pltpu.CompilerParams(use_tc_tiling_on_sc=False),
        ))
    return _gather_cache[0]


_BB = 2048
_NBLK = _BATCH // _BB


def _tower_body(ue, ie, uW1, ub1, uW2, ub2, iW1, ib1, iW2, ib2, out):
    u = jnp.maximum(
        jnp.dot(ue[...], uW1[...], preferred_element_type=jnp.float32)
        + ub1[...], 0.0)
    u = jnp.maximum(
        jnp.dot(u, uW2[...], preferred_element_type=jnp.float32)
        + ub2[...], 0.0)
    v = jnp.maximum(
        jnp.dot(ie[...], iW1[...], preferred_element_type=jnp.float32)
        + ib1[...], 0.0)
    v = jnp.maximum(
        jnp.dot(v, iW2[...], preferred_element_type=jnp.float32)
        + ib2[...], 0.0)
    un = jnp.sqrt(jnp.sum(u * u, axis=-1, keepdims=True))
    vn = jnp.sqrt(jnp.sum(v * v, axis=-1, keepdims=True))
    dot = jnp.sum(u * v, axis=-1, keepdims=True)
    out[...] = dot / (jnp.maximum(un, 1e-12) * jnp.maximum(vn, 1e-12))


def _towers(ue, ie, uW1, ub1, uW2, ub2, iW1, ib1, iW2, ib2):
    full = lambda shape: pl.BlockSpec(shape, lambda b: (0, 0))
    return pl.pallas_call(
        _tower_body,
        grid=(_NBLK,),
        in_specs=[
            pl.BlockSpec((_BB, _EMB), lambda b: (b, 0)),
            pl.BlockSpec((_BB, _EMB), lambda b: (b, 0)),
            full(uW1.shape), full(ub1.shape), full(uW2.shape), full(ub2.shape),
            full(iW1.shape), full(ib1.shape), full(iW2.shape), full(ib2.shape),
        ],
        out_specs=pl.BlockSpec((_BB, 1), lambda b: (b, 0)),
        out_shape=jax.ShapeDtypeStruct((_BATCH, 1), jnp.float32),
    )(ue, ie, uW1, ub1, uW2, ub2, iW1, ib1, iW2, ib2)


def kernel(user_ids, item_ids, user_table, item_table,
           uW1, ub1, uW2, ub2, iW1, ib1, iW2, ib2):
    ue, ie = _gather_kernel()(user_table, item_table, user_ids, item_ids)
    scores = _towers(ue, ie,
                     uW1, ub1.reshape(1, -1), uW2, ub2.reshape(1, -1),
                     iW1, ib1.reshape(1, -1), iW2, ib2.reshape(1, -1))
    return scores.reshape(_BATCH)


# speedup vs baseline: 2.3801x; 2.3801x over previous
import jax
import jax.numpy as jnp
from jax import lax
from jax.experimental import pallas as pl
from jax.experimental.pallas import tpu as pltpu
from jax.experimental.pallas import tpu_sc as plsc

_BATCH = 16384
_VOCAB = 1000000
_EMB = 64
_NC = 2
_NS = 16
_NW = _NC * _NS
_BPW = _BATCH // _NW
_Q = 128
_NQ = _BPW // _Q

_BV = 8192
_NV = -(-_VOCAB // _BV)
_HB = _BV // 2
_PROWS = _NV * _HB


def _pack_body(ut, it, uo, io):
    x = ut[...]
    uo[...] = jnp.concatenate([x[:, 0:_HB].T, x[:, _HB:_BV].T], axis=1)
    y = it[...]
    io[...] = jnp.concatenate([y[:, 0:_HB].T, y[:, _HB:_BV].T], axis=1)


def _pack(utT, itT):
    return pl.pallas_call(
        _pack_body,
        grid=(_NV,),
        in_specs=[
            pl.BlockSpec((_EMB, _BV), lambda v: (0, v)),
            pl.BlockSpec((_EMB, _BV), lambda v: (0, v)),
        ],
        out_specs=[
            pl.BlockSpec((_HB, 128), lambda v: (v, 0)),
            pl.BlockSpec((_HB, 128), lambda v: (v, 0)),
        ],
        out_shape=[
            jax.ShapeDtypeStruct((_PROWS, 128), jnp.float32),
            jax.ShapeDtypeStruct((_PROWS, 128), jnp.float32),
        ],
    )(utT, itT)


def _gather_body(pu, pi, user_ids, item_ids, u_out, i_out,
                 uids_v, iids_v, uidx, iidx, bufs, sem):
    wid = lax.axis_index("s") * _NC + lax.axis_index("c")
    base = wid * _BPW
    pltpu.sync_copy(user_ids.at[pl.ds(base, _BPW)], uids_v)
    pltpu.sync_copy(item_ids.at[pl.ds(base, _BPW)], iids_v)
    for q in range(_NQ):
        for g in range(_Q // 16):
            u = uids_v[pl.ds(q * _Q + g * 16, 16)]
            uidx[q, pl.ds(g * 16, 16)] = ((u >> 13) << 12) | (u & 4095)
            i = iids_v[pl.ds(q * _Q + g * 16, 16)]
            iidx[q, pl.ds(g * 16, 16)] = ((i >> 13) << 12) | (i & 4095)
    prev = None
    for q in range(_NQ):
        s = q % 2
        cu = pltpu.async_copy(pu.at[uidx.at[q]], bufs.at[2 * s], sem)
        ci = pltpu.async_copy(pi.at[iidx.at[q]], bufs.at[2 * s + 1], sem)
        if prev is not None:
            pcu, pci, pq = prev
            ps = pq % 2
            pcu.wait()
            pltpu.sync_copy(bufs.at[2 * ps],
                            u_out.at[pl.ds(base + pq * _Q, _Q)])
            pci.wait()
            pltpu.sync_copy(bufs.at[2 * ps + 1],
                            i_out.at[pl.ds(base + pq * _Q, _Q)])
        prev = (cu, ci, q)
    pcu, pci, pq = prev
    ps = pq % 2
    pcu.wait()
    pltpu.sync_copy(bufs.at[2 * ps], u_out.at[pl.ds(base + pq * _Q, _Q)])
    pci.wait()
    pltpu.sync_copy(bufs.at[2 * ps + 1], i_out.at[pl.ds(base + pq * _Q, _Q)])


_gather_cache = []


def _gather_kernel():
    if not _gather_cache:
        _gather_cache.append(pl.kernel(
            _gather_body,
            out_type=[jax.ShapeDtypeStruct((_BATCH, 128), jnp.float32),
                      jax.ShapeDtypeStruct((_BATCH, 128), jnp.float32)],
            mesh=plsc.VectorSubcoreMesh(core_axis_name="c",
                                        subcore_axis_name="s",
                                        num_cores=_NC, num_subcores=_NS),
            scratch_types=[
                pltpu.VMEM((_BPW,), jnp.int32),
                pltpu.VMEM((_BPW,), jnp.int32),
                pltpu.VMEM((_NQ, _Q), jnp.int32),
                pltpu.VMEM((_NQ, _Q), jnp.int32),
                pltpu.VMEM((4, _Q, 128), jnp.float32),
                pltpu.SemaphoreType.DMA,
            ],
        ))
    return _gather_cache[0]


_BB = 2048
_NBLK = _BATCH // _BB


def _tower_body(ur, ir, uid, iid, uW1, ub1, uW2, ub2, iW1, ib1, iW2, ib2,
                out):
    upar = ((uid[...] >> 12) & 1) == 1
    ipar = ((iid[...] >> 12) & 1) == 1
    ue = jnp.where(upar, ur[:, 64:128], ur[:, 0:64])
    ie = jnp.where(ipar, ir[:, 64:128], ir[:, 0:64])
    u = jnp.maximum(
        jnp.dot(ue, uW1[...], preferred_element_type=jnp.float32)
        + ub1[...], 0.0)
    u = jnp.maximum(
        jnp.dot(u, uW2[...], preferred_element_type=jnp.float32)
        + ub2[...], 0.0)
    v = jnp.maximum(
        jnp.dot(ie, iW1[...], preferred_element_type=jnp.float32)
        + ib1[...], 0.0)
    v = jnp.maximum(
        jnp.dot(v, iW2[...], preferred_element_type=jnp.float32)
        + ib2[...], 0.0)
    un = jnp.sqrt(jnp.sum(u * u, axis=-1, keepdims=True))
    vn = jnp.sqrt(jnp.sum(v * v, axis=-1, keepdims=True))
    dot = jnp.sum(u * v, axis=-1, keepdims=True)
    out[...] = dot / (jnp.maximum(un, 1e-12) * jnp.maximum(vn, 1e-12))


def _towers(ur, ir, uid2, iid2, uW1, ub1, uW2, ub2, iW1, ib1, iW2, ib2):
    full = lambda shape: pl.BlockSpec(shape, lambda b: (0, 0))
    return pl.pallas_call(
        _tower_body,
        grid=(_NBLK,),
        in_specs=[
            pl.BlockSpec((_BB, 128), lambda b: (b, 0)),
            pl.BlockSpec((_BB, 128), lambda b: (b, 0)),
            pl.BlockSpec((_BB, 1), lambda b: (b, 0)),
            pl.BlockSpec((_BB, 1), lambda b: (b, 0)),
            full(uW1.shape), full(ub1.shape), full(uW2.shape), full(ub2.shape),
            full(iW1.shape), full(ib1.shape), full(iW2.shape), full(ib2.shape),
        ],
        out_specs=pl.BlockSpec((_BB, 1), lambda b: (b, 0)),
        out_shape=jax.ShapeDtypeStruct((_BATCH, 1), jnp.float32),
    )(ur, ir, uid2, iid2, uW1, ub1, uW2, ub2, iW1, ib1, iW2, ib2)


def kernel(user_ids, item_ids, user_table, item_table,
           uW1, ub1, uW2, ub2, iW1, ib1, iW2, ib2):
    pu, pi = _pack(user_table.T, item_table.T)
    ur, ir = _gather_kernel()(pu, pi, user_ids, item_ids)
    scores = _towers(ur, ir,
                     user_ids.reshape(-1, 1), item_ids.reshape(-1, 1),
                     uW1, ub1.reshape(1, -1), uW2, ub2.reshape(1, -1),
                     iW1, ib1.reshape(1, -1), iW2, ib2.reshape(1, -1))
    return scores.reshape(_BATCH)


# speedup vs baseline: 2.4167x; 1.0154x over previous
import jax
import jax.numpy as jnp
from jax import lax
from jax.experimental import pallas as pl
from jax.experimental.pallas import tpu as pltpu
from jax.experimental.pallas import tpu_sc as plsc

_BATCH = 16384
_VOCAB = 1000000
_EMB = 64
_NC = 2
_NS = 16
_NW = _NC * _NS
_BPW = _BATCH // _NW
_Q = 128
_NQ = _BPW // _Q

_BV = 16384
_NV = -(-_VOCAB // _BV)
_HB = _BV // 2
_PROWS = _NV * _HB
_SHB = _BV.bit_length() - 1
_SHH = _SHB - 1
_HMASK = _HB - 1


def _pack_body(ut, it, uo, io):
    x = ut[...]
    uo[...] = jnp.concatenate([x[:, 0:_HB].T, x[:, _HB:_BV].T], axis=1)
    y = it[...]
    io[...] = jnp.concatenate([y[:, 0:_HB].T, y[:, _HB:_BV].T], axis=1)


def _pack(utT, itT):
    return pl.pallas_call(
        _pack_body,
        grid=(_NV,),
        in_specs=[
            pl.BlockSpec((_EMB, _BV), lambda v: (0, v)),
            pl.BlockSpec((_EMB, _BV), lambda v: (0, v)),
        ],
        out_specs=[
            pl.BlockSpec((_HB, 128), lambda v: (v, 0)),
            pl.BlockSpec((_HB, 128), lambda v: (v, 0)),
        ],
        out_shape=[
            jax.ShapeDtypeStruct((_PROWS, 128), jnp.float32),
            jax.ShapeDtypeStruct((_PROWS, 128), jnp.float32),
        ],
    )(utT, itT)


def _gather_body(pu, pi, user_ids, item_ids, u_out, i_out,
                 uids_v, iids_v, uidx, iidx, bufs, sem):
    wid = lax.axis_index("s") * _NC + lax.axis_index("c")
    base = wid * _BPW
    pltpu.sync_copy(user_ids.at[pl.ds(base, _BPW)], uids_v)
    pltpu.sync_copy(item_ids.at[pl.ds(base, _BPW)], iids_v)
    for q in range(_NQ):
        for g in range(_Q // 16):
            u = uids_v[pl.ds(q * _Q + g * 16, 16)]
            uidx[q, pl.ds(g * 16, 16)] = ((u >> _SHB) << _SHH) | (u & _HMASK)
            i = iids_v[pl.ds(q * _Q + g * 16, 16)]
            iidx[q, pl.ds(g * 16, 16)] = ((i >> _SHB) << _SHH) | (i & _HMASK)
    prev = None
    for q in range(_NQ):
        s = q % 2
        cu = pltpu.async_copy(pu.at[uidx.at[q]], bufs.at[2 * s], sem)
        ci = pltpu.async_copy(pi.at[iidx.at[q]], bufs.at[2 * s + 1], sem)
        if prev is not None:
            pcu, pci, pq = prev
            ps = pq % 2
            pcu.wait()
            pltpu.sync_copy(bufs.at[2 * ps],
                            u_out.at[pl.ds(base + pq * _Q, _Q)])
            pci.wait()
            pltpu.sync_copy(bufs.at[2 * ps + 1],
                            i_out.at[pl.ds(base + pq * _Q, _Q)])
        prev = (cu, ci, q)
    pcu, pci, pq = prev
    ps = pq % 2
    pcu.wait()
    pltpu.sync_copy(bufs.at[2 * ps], u_out.at[pl.ds(base + pq * _Q, _Q)])
    pci.wait()
    pltpu.sync_copy(bufs.at[2 * ps + 1], i_out.at[pl.ds(base + pq * _Q, _Q)])


_gather_cache = []


def _gather_kernel():
    if not _gather_cache:
        _gather_cache.append(pl.kernel(
            _gather_body,
            out_type=[jax.ShapeDtypeStruct((_BATCH, 128), jnp.float32),
                      jax.ShapeDtypeStruct((_BATCH, 128), jnp.float32)],
            mesh=plsc.VectorSubcoreMesh(core_axis_name="c",
                                        subcore_axis_name="s",
                                        num_cores=_NC, num_subcores=_NS),
            scratch_types=[
                pltpu.VMEM((_BPW,), jnp.int32),
                pltpu.VMEM((_BPW,), jnp.int32),
                pltpu.VMEM((_NQ, _Q), jnp.int32),
                pltpu.VMEM((_NQ, _Q), jnp.int32),
                pltpu.VMEM((4, _Q, 128), jnp.float32),
                pltpu.SemaphoreType.DMA,
            ],
        ))
    return _gather_cache[0]


_BB = 2048
_NBLK = _BATCH // _BB


def _tower_body(ur, ir, uid, iid, uW1, ub1, uW2, ub2, iW1, ib1, iW2, ib2,
                out):
    upar = ((uid[...] >> _SHH) & 1) == 1
    ipar = ((iid[...] >> _SHH) & 1) == 1
    ue = jnp.where(upar, ur[:, 64:128], ur[:, 0:64])
    ie = jnp.where(ipar, ir[:, 64:128], ir[:, 0:64])
    u = jnp.maximum(
        jnp.dot(ue, uW1[...], preferred_element_type=jnp.float32)
        + ub1[...], 0.0)
    u = jnp.maximum(
        jnp.dot(u, uW2[...], preferred_element_type=jnp.float32)
        + ub2[...], 0.0)
    v = jnp.maximum(
        jnp.dot(ie, iW1[...], preferred_element_type=jnp.float32)
        + ib1[...], 0.0)
    v = jnp.maximum(
        jnp.dot(v, iW2[...], preferred_element_type=jnp.float32)
        + ib2[...], 0.0)
    un = jnp.sqrt(jnp.sum(u * u, axis=-1, keepdims=True))
    vn = jnp.sqrt(jnp.sum(v * v, axis=-1, keepdims=True))
    dot = jnp.sum(u * v, axis=-1, keepdims=True)
    out[...] = dot / (jnp.maximum(un, 1e-12) * jnp.maximum(vn, 1e-12))


def _towers(ur, ir, uid2, iid2, uW1, ub1, uW2, ub2, iW1, ib1, iW2, ib2):
    full = lambda shape: pl.BlockSpec(shape, lambda b: (0, 0))
    return pl.pallas_call(
        _tower_body,
        grid=(_NBLK,),
        in_specs=[
            pl.BlockSpec((_BB, 128), lambda b: (b, 0)),
            pl.BlockSpec((_BB, 128), lambda b: (b, 0)),
            pl.BlockSpec((_BB, 1), lambda b: (b, 0)),
            pl.BlockSpec((_BB, 1), lambda b: (b, 0)),
            full(uW1.shape), full(ub1.shape), full(uW2.shape), full(ub2.shape),
            full(iW1.shape), full(ib1.shape), full(iW2.shape), full(ib2.shape),
        ],
        out_specs=pl.BlockSpec((_BB, 1), lambda b: (b, 0)),
        out_shape=jax.ShapeDtypeStruct((_BATCH, 1), jnp.float32),
    )(ur, ir, uid2, iid2, uW1, ub1, uW2, ub2, iW1, ib1, iW2, ib2)


def kernel(user_ids, item_ids, user_table, item_table,
           uW1, ub1, uW2, ub2, iW1, ib1, iW2, ib2):
    pu, pi = _pack(user_table.T, item_table.T)
    ur, ir = _gather_kernel()(pu, pi, user_ids, item_ids)
    scores = _towers(ur, ir,
                     user_ids.reshape(-1, 1), item_ids.reshape(-1, 1),
                     uW1, ub1.reshape(1, -1), uW2, ub2.reshape(1, -1),
                     iW1, ib1.reshape(1, -1), iW2, ib2.reshape(1, -1))
    return scores.reshape(_BATCH)


# speedup vs baseline: 3.3897x; 1.4026x over previous
import jax
import jax.numpy as jnp
from jax import lax
from jax.experimental import pallas as pl
from jax.experimental.pallas import tpu as pltpu
from jax.experimental.pallas import tpu_sc as plsc

_BATCH = 16384
_VOCAB = 1000000
_EMB = 64
_NC = 2
_NS = 16
_NW = _NC * _NS
_BPW = _BATCH // _NW
_Q = 128
_NQ = _BPW // _Q

_BV = 16384
_NV = -(-_VOCAB // _BV)
_QB = _BV // 4
_PROWS = _NV * _QB
_SHB = _BV.bit_length() - 1
_SHQ = _QB.bit_length() - 1
_QMASK = _QB - 1


def _pack_body(ut, it, uo, io):
    for ref, o in ((ut, uo), (it, io)):
        x = ref[...]
        words = []
        for t in (0, 2):
            a = x[:, t * _QB:(t + 1) * _QB].T.astype(jnp.bfloat16)
            b = x[:, (t + 1) * _QB:(t + 2) * _QB].T.astype(jnp.bfloat16)
            a32 = lax.bitcast_convert_type(a, jnp.uint16).astype(jnp.uint32)
            b32 = lax.bitcast_convert_type(b, jnp.uint16).astype(jnp.uint32)
            words.append(a32 | (b32 << 16))
        o[...] = lax.bitcast_convert_type(
            jnp.concatenate(words, axis=1), jnp.int32)


def _pack(utT, itT):
    return pl.pallas_call(
        _pack_body,
        grid=(_NV,),
        in_specs=[
            pl.BlockSpec((_EMB, _BV), lambda v: (0, v)),
            pl.BlockSpec((_EMB, _BV), lambda v: (0, v)),
        ],
        out_specs=[
            pl.BlockSpec((_QB, 128), lambda v: (v, 0)),
            pl.BlockSpec((_QB, 128), lambda v: (v, 0)),
        ],
        out_shape=[
            jax.ShapeDtypeStruct((_PROWS, 128), jnp.int32),
            jax.ShapeDtypeStruct((_PROWS, 128), jnp.int32),
        ],
    )(utT, itT)


def _gather_body(pu, pi, user_ids, item_ids, u_out, i_out,
                 uids_v, iids_v, uidx, iidx, bufs, sem):
    wid = lax.axis_index("s") * _NC + lax.axis_index("c")
    base = wid * _BPW
    pltpu.sync_copy(user_ids.at[pl.ds(base, _BPW)], uids_v)
    pltpu.sync_copy(item_ids.at[pl.ds(base, _BPW)], iids_v)
    for q in range(_NQ):
        for g in range(_Q // 16):
            u = uids_v[pl.ds(q * _Q + g * 16, 16)]
            uidx[q, pl.ds(g * 16, 16)] = ((u >> _SHB) << _SHQ) | (u & _QMASK)
            i = iids_v[pl.ds(q * _Q + g * 16, 16)]
            iidx[q, pl.ds(g * 16, 16)] = ((i >> _SHB) << _SHQ) | (i & _QMASK)
    prev = None
    for q in range(_NQ):
        s = q % 2
        cu = pltpu.async_copy(pu.at[uidx.at[q]], bufs.at[2 * s], sem)
        ci = pltpu.async_copy(pi.at[iidx.at[q]], bufs.at[2 * s + 1], sem)
        if prev is not None:
            pcu, pci, pq = prev
            ps = pq % 2
            pcu.wait()
            pltpu.sync_copy(bufs.at[2 * ps],
                            u_out.at[pl.ds(base + pq * _Q, _Q)])
            pci.wait()
            pltpu.sync_copy(bufs.at[2 * ps + 1],
                            i_out.at[pl.ds(base + pq * _Q, _Q)])
        prev = (cu, ci, q)
    pcu, pci, pq = prev
    ps = pq % 2
    pcu.wait()
    pltpu.sync_copy(bufs.at[2 * ps], u_out.at[pl.ds(base + pq * _Q, _Q)])
    pci.wait()
    pltpu.sync_copy(bufs.at[2 * ps + 1], i_out.at[pl.ds(base + pq * _Q, _Q)])


_gather_cache = []


def _gather_kernel():
    if not _gather_cache:
        _gather_cache.append(pl.kernel(
            _gather_body,
            out_type=[jax.ShapeDtypeStruct((_BATCH, 128), jnp.int32),
                      jax.ShapeDtypeStruct((_BATCH, 128), jnp.int32)],
            mesh=plsc.VectorSubcoreMesh(core_axis_name="c",
                                        subcore_axis_name="s",
                                        num_cores=_NC, num_subcores=_NS),
            scratch_types=[
                pltpu.VMEM((_BPW,), jnp.int32),
                pltpu.VMEM((_BPW,), jnp.int32),
                pltpu.VMEM((_NQ, _Q), jnp.int32),
                pltpu.VMEM((_NQ, _Q), jnp.int32),
                pltpu.VMEM((4, _Q, 128), jnp.int32),
                pltpu.SemaphoreType.DMA,
            ],
        ))
    return _gather_cache[0]


_BB = 2048
_NBLK = _BATCH // _BB


def _tower_body(ur, ir, uid, iid, uW1, ub1, uW2, ub2, iW1, ib1, iW2, ib2,
                out):
    us = (uid[...] >> _SHQ) & 3
    its = (iid[...] >> _SHQ) & 3
    uw = jnp.where(us >= 2, ur[:, 64:128], ur[:, 0:64])
    iw = jnp.where(its >= 2, ir[:, 64:128], ir[:, 0:64])
    ue = lax.bitcast_convert_type(
        jnp.where((us & 1) == 1, uw & jnp.int32(-65536), uw << 16),
        jnp.float32)
    ie = lax.bitcast_convert_type(
        jnp.where((its & 1) == 1, iw & jnp.int32(-65536), iw << 16),
        jnp.float32)
    u = jnp.maximum(
        jnp.dot(ue, uW1[...], preferred_element_type=jnp.float32)
        + ub1[...], 0.0)
    u = jnp.maximum(
        jnp.dot(u, uW2[...], preferred_element_type=jnp.float32)
        + ub2[...], 0.0)
    v = jnp.maximum(
        jnp.dot(ie, iW1[...], preferred_element_type=jnp.float32)
        + ib1[...], 0.0)
    v = jnp.maximum(
        jnp.dot(v, iW2[...], preferred_element_type=jnp.float32)
        + ib2[...], 0.0)
    un = jnp.sqrt(jnp.sum(u * u, axis=-1, keepdims=True))
    vn = jnp.sqrt(jnp.sum(v * v, axis=-1, keepdims=True))
    dot = jnp.sum(u * v, axis=-1, keepdims=True)
    out[...] = dot / (jnp.maximum(un, 1e-12) * jnp.maximum(vn, 1e-12))


def _towers(ur, ir, uid2, iid2, uW1, ub1, uW2, ub2, iW1, ib1, iW2, ib2):
    full = lambda shape: pl.BlockSpec(shape, lambda b: (0, 0))
    return pl.pallas_call(
        _tower_body,
        grid=(_NBLK,),
        in_specs=[
            pl.BlockSpec((_BB, 128), lambda b: (b, 0)),
            pl.BlockSpec((_BB, 128), lambda b: (b, 0)),
            pl.BlockSpec((_BB, 1), lambda b: (b, 0)),
            pl.BlockSpec((_BB, 1), lambda b: (b, 0)),
            full(uW1.shape), full(ub1.shape), full(uW2.shape), full(ub2.shape),
            full(iW1.shape), full(ib1.shape), full(iW2.shape), full(ib2.shape),
        ],
        out_specs=pl.BlockSpec((_BB, 1), lambda b: (b, 0)),
        out_shape=jax.ShapeDtypeStruct((_BATCH, 1), jnp.float32),
    )(ur, ir, uid2, iid2, uW1, ub1, uW2, ub2, iW1, ib1, iW2, ib2)


def kernel(user_ids, item_ids, user_table, item_table,
           uW1, ub1, uW2, ub2, iW1, ib1, iW2, ib2):
    pu, pi = _pack(user_table.T, item_table.T)
    ur, ir = _gather_kernel()(pu, pi, user_ids, item_ids)
    scores = _towers(ur, ir,
                     user_ids.reshape(-1, 1), item_ids.reshape(-1, 1),
                     uW1, ub1.reshape(1, -1), uW2, ub2.reshape(1, -1),
                     iW1, ib1.reshape(1, -1), iW2, ib2.reshape(1, -1))
    return scores.reshape(_BATCH)
